# baseline (speedup 1.0000x reference)
import functools

import jax
import jax.numpy as jnp
from jax import lax
from jax.experimental import pallas as pl
from jax.experimental.pallas import tpu as pltpu
from jax.experimental.pallas import tpu_sc as plsc

B = 16384
K = 16

_INFO = plsc.get_sparse_core_info()
_NC, _NS, _L = _INFO.num_cores, _INFO.num_subcores, _INFO.num_lanes
_NW = _NC * _NS
_BPW = B // _NW
_CH = 128
_NCH = _BPW // _CH
_NG = _BPW // _L

_BITREV = [((j & 1) << 3) | ((j & 2) << 1) | ((j & 4) >> 1) | ((j & 8) >> 3)
           for j in range(16)]


def _sc_body(users_hbm, items_hbm, gu_hbm, gi_hbm,
             xui_out, gu_out, gi_out,
             uidx_v, iidx_v, urows_v, irows_v, xui_v, gsem, osem):
    wid = lax.axis_index("s") * _NC + lax.axis_index("c")
    base = wid * _BPW

    pltpu.sync_copy(users_hbm.at[pl.ds(base, _BPW)], uidx_v)
    pltpu.sync_copy(items_hbm.at[pl.ds(base, _BPW)], iidx_v)

    gathers = []
    for c in range(_NCH):
        sl = pl.ds(c * _CH, _CH)
        gathers.append(pltpu.async_copy(
            gu_hbm.at[uidx_v.at[sl]], urows_v.at[sl, :], gsem))
        gathers.append(pltpu.async_copy(
            gi_hbm.at[iidx_v.at[sl]], irows_v.at[sl, :], gsem))
    for cp in gathers:
        cp.wait()

    out_u = pltpu.async_copy(urows_v, gu_out.at[pl.ds(base, _BPW), :], osem)
    out_i = pltpu.async_copy(irows_v, gi_out.at[pl.ds(base, _BPW), :], osem)

    lanes = lax.iota(jnp.int32, _L)
    stages = [((lanes & d) == 0, lanes ^ d) for d in (8, 4, 2, 1)]

    def group(g, carry):
        r0 = g * _L
        vecs = []
        for j in range(_L):
            r = r0 + _BITREV[j]
            vecs.append(urows_v[r, :] * irows_v[r, :])
        for m, sw in stages:
            nxt = []
            for a, b in zip(vecs[0::2], vecs[1::2]):
                nxt.append(jnp.where(m, a, b) +
                           jnp.where(m, b, a).at[sw].get(
                               mode="promise_in_bounds", unique_indices=True))
            vecs = nxt
        xui_v[pl.ds(r0, _L)] = vecs[0]
        return carry

    lax.fori_loop(0, _NG, group, 0)

    pltpu.sync_copy(xui_v, xui_out.at[pl.ds(base, _BPW)])
    out_u.wait()
    out_i.wait()


_mf_kernel = functools.partial(
    pl.kernel,
    mesh=plsc.VectorSubcoreMesh(core_axis_name="c", subcore_axis_name="s"),
    out_type=(
        jax.ShapeDtypeStruct((B,), jnp.float32),
        jax.ShapeDtypeStruct((B, K), jnp.float32),
        jax.ShapeDtypeStruct((B, K), jnp.float32),
    ),
    scratch_types=[
        pltpu.VMEM((_BPW,), jnp.int32),
        pltpu.VMEM((_BPW,), jnp.int32),
        pltpu.VMEM((_BPW, K), jnp.float32),
        pltpu.VMEM((_BPW, K), jnp.float32),
        pltpu.VMEM((_BPW,), jnp.float32),
        pltpu.SemaphoreType.DMA,
        pltpu.SemaphoreType.DMA,
    ],
    compiler_params=pltpu.CompilerParams(use_tc_tiling_on_sc=False),
)(_sc_body)


def kernel(users, items, Gu, Gi):
    xui, gamma_u, gamma_i = _mf_kernel(
        users.astype(jnp.int32), items.astype(jnp.int32), Gu, Gi)
    return (xui, gamma_u, gamma_i)
